# Initial kernel scaffold; baseline (speedup 1.0000x reference)
import functools

import jax
import jax.numpy as jnp
from jax import lax
from jax.experimental import pallas as pl
from jax.experimental.pallas import tpu as pltpu
from jax.experimental.pallas import tpu_sc as plsc

NC = 2
NS = 16
NW = NC * NS
CHUNK = 128


def _fill_f32(ref, nrows, ncols, value):
    v = jnp.full((16,), value, jnp.float32)

    @pl.loop(0, nrows)
    def _(i):
        for k in range(ncols // 16):
            ref[i, pl.ds(k * 16, 16)] = v


def _make_deg_kernel(N, NACC, K):
    mesh = plsc.VectorSubcoreMesh(core_axis_name="c", subcore_axis_name="s")
    zrows = NACC // NS
    orows = N // NS

    @functools.partial(
        pl.kernel,
        out_type=jax.ShapeDtypeStruct((NC, N, 16), jnp.float32),
        mesh=mesh,
        scratch_types=[
            pltpu.VMEM((K, CHUNK), jnp.int32),
            pltpu.VMEM((CHUNK, 16), jnp.float32),
            pltpu.VMEM_SHARED((NACC, 16), jnp.float32),
        ],
    )
    def deg_kernel(dst_hbm, out_hbm, dst_v, ones_v, acc_sh):
        c = lax.axis_index("c")
        s = lax.axis_index("s")
        wid = s * NC + c
        _fill_f32(ones_v, CHUNK, 16, 0.0)
        zb = s * zrows
        for off in range(0, zrows, CHUNK):
            n = min(CHUNK, zrows - off)
            pltpu.sync_copy(ones_v.at[pl.ds(0, n)], acc_sh.at[pl.ds(zb + off, n)])
        _fill_f32(ones_v, CHUNK, 16, 1.0)
        pltpu.sync_copy(dst_hbm.at[wid], dst_v)
        plsc.subcore_barrier()

        @pl.loop(0, K)
        def _(j):
            pltpu.sync_copy(ones_v, acc_sh.at[dst_v.at[j]], add=True)

        plsc.subcore_barrier()
        ob = s * orows
        for off in range(0, orows, CHUNK):
            n = min(CHUNK, orows - off)
            pltpu.sync_copy(acc_sh.at[pl.ds(ob + off, n)],
                            out_hbm.at[c, pl.ds(ob + off, n)])

    return deg_kernel


def _make_agg_kernel(N, NACC, K, D):
    mesh = plsc.VectorSubcoreMesh(core_axis_name="c", subcore_axis_name="s")
    zrows = NACC // NS
    orows = N // NS

    @functools.partial(
        pl.kernel,
        out_type=jax.ShapeDtypeStruct((NC, N, D), jnp.float32),
        mesh=mesh,
        scratch_types=[
            pltpu.VMEM((K, CHUNK), jnp.int32),
            pltpu.VMEM((K, CHUNK), jnp.int32),
            pltpu.VMEM((CHUNK, D), jnp.float32),
            pltpu.VMEM_SHARED((NACC, D), jnp.float32),
            pltpu.SemaphoreType.DMA,
        ],
    )
    def agg_kernel(table_hbm, src_hbm, dst_hbm, out_hbm,
                   src_v, dst_v, rows_v, acc_sh, sem):
        c = lax.axis_index("c")
        s = lax.axis_index("s")
        wid = s * NC + c
        _fill_f32(rows_v, CHUNK, D, 0.0)
        zb = s * zrows
        for off in range(0, zrows, CHUNK):
            n = min(CHUNK, zrows - off)
            pltpu.sync_copy(rows_v.at[pl.ds(0, n)], acc_sh.at[pl.ds(zb + off, n)])
        pltpu.sync_copy(src_hbm.at[wid], src_v)
        pltpu.sync_copy(dst_hbm.at[wid], dst_v)
        plsc.subcore_barrier()

        @pl.loop(0, K)
        def _(j):
            pltpu.async_copy(table_hbm.at[src_v.at[j]], rows_v, sem).wait()
            pltpu.sync_copy(rows_v, acc_sh.at[dst_v.at[j]], add=True)

        plsc.subcore_barrier()
        ob = s * orows
        for off in range(0, orows, CHUNK):
            n = min(CHUNK, orows - off)
            pltpu.sync_copy(acc_sh.at[pl.ds(ob + off, n)],
                            out_hbm.at[c, pl.ds(ob + off, n)])

    return agg_kernel


def _mm_body(x_ref, w_ref, o_ref):
    o_ref[...] = jnp.dot(x_ref[...], w_ref[...],
                         preferred_element_type=jnp.float32)


def _dinv(d0_ref, d1_ref):
    deg = d0_ref[:, :1] + d1_ref[:, :1] + 1.0
    return lax.rsqrt(deg)


def _scale_body(z_ref, d0_ref, d1_ref, o_ref):
    o_ref[...] = z_ref[...] * _dinv(d0_ref, d1_ref)


def _layer2_body(a0_ref, a1_ref, p_ref, d0_ref, d1_ref, w_ref, b_ref, o_ref):
    dinv = _dinv(d0_ref, d1_ref)
    h = (a0_ref[...] + a1_ref[...] + p_ref[...]) * dinv + b_ref[...]
    h = jnp.maximum(h, 0.0)
    o_ref[...] = jnp.dot(h, w_ref[...],
                         preferred_element_type=jnp.float32) * dinv


def _final_body(a0_ref, a1_ref, p_ref, d0_ref, d1_ref, b_ref, o_ref):
    dinv = _dinv(d0_ref, d1_ref)
    o_ref[...] = (a0_ref[...] + a1_ref[...] + p_ref[...]) * dinv + b_ref[...]


def kernel(x, edge_index, W1, b1, W2, b2):
    N, D = x.shape
    E = edge_index.shape[1]
    src = edge_index[0].astype(jnp.int32)
    dst = edge_index[1].astype(jnp.int32)

    per_op = NW * CHUNK
    K = -(-E // per_op)
    padn = K * per_op - E
    NACC = N + 16
    src3 = jnp.concatenate([src, jnp.zeros((padn,), jnp.int32)]).reshape(NW, K, CHUNK)
    dst3 = jnp.concatenate([dst, jnp.full((padn,), N, jnp.int32)]).reshape(NW, K, CHUNK)

    deg_k = _make_deg_kernel(N, NACC, K)
    agg_k = _make_agg_kernel(N, NACC, K, D)

    degp = deg_k(dst3)
    d0, d1 = degp[0], degp[1]

    BR = 400
    G = N // BR
    fullW = pl.BlockSpec((D, D), lambda i: (0, 0))
    fullB = pl.BlockSpec((1, D), lambda i: (0, 0))
    rowD = lambda: pl.BlockSpec((BR, D), lambda i: (i, 0))
    rowd = lambda: pl.BlockSpec((BR, 16), lambda i: (i, 0))
    outD = jax.ShapeDtypeStruct((N, D), jnp.float32)
    b1r = b1.reshape(1, D)
    b2r = b2.reshape(1, D)

    z1 = pl.pallas_call(
        _mm_body, grid=(G,),
        in_specs=[rowD(), fullW],
        out_specs=rowD(), out_shape=outD)(x, W1)

    p1 = pl.pallas_call(
        _scale_body, grid=(G,),
        in_specs=[rowD(), rowd(), rowd()],
        out_specs=rowD(), out_shape=outD)(z1, d0, d1)

    a1 = agg_k(p1, src3, dst3)

    p2 = pl.pallas_call(
        _layer2_body, grid=(G,),
        in_specs=[rowD(), rowD(), rowD(), rowd(), rowd(), fullW, fullB],
        out_specs=rowD(), out_shape=outD)(a1[0], a1[1], p1, d0, d1, W2, b1r)

    a2 = agg_k(p2, src3, dst3)

    out = pl.pallas_call(
        _final_body, grid=(G,),
        in_specs=[rowD(), rowD(), rowD(), rowd(), rowd(), fullB],
        out_specs=rowD(), out_shape=outD)(a2[0], a2[1], p2, d0, d1, b2r)

    return out


# speedup vs baseline: 13.2205x; 13.2205x over previous
import functools

import jax
import jax.numpy as jnp
from jax import lax
from jax.experimental import pallas as pl
from jax.experimental.pallas import tpu as pltpu
from jax.experimental.pallas import tpu_sc as plsc

NC = 2
NS = 16
NW = NC * NS
CHUNK = 128


def _fill_f32(ref, nrows, ncols, value):
    v = jnp.full((16,), value, jnp.float32)

    @pl.loop(0, nrows)
    def _(i):
        for k in range(ncols // 16):
            ref[i, pl.ds(k * 16, 16)] = v


def _make_deg_kernel(N, NACC, K):
    mesh = plsc.VectorSubcoreMesh(core_axis_name="c", subcore_axis_name="s")
    zrows = NACC // NS

    @functools.partial(
        pl.kernel,
        out_type=jax.ShapeDtypeStruct((NC, NACC, 16), jnp.float32),
        mesh=mesh,
        scratch_types=[
            pltpu.VMEM((K, CHUNK), jnp.int32),
            pltpu.VMEM((CHUNK, 16), jnp.float32),
            pltpu.VMEM_SHARED((NACC, 16), jnp.float32),
        ],
    )
    def deg_kernel(dst_hbm, out_hbm, dst_v, ones_v, acc_sh):
        c = lax.axis_index("c")
        s = lax.axis_index("s")
        wid = s * NC + c
        _fill_f32(ones_v, CHUNK, 16, 0.0)
        zb = s * zrows
        for off in range(0, zrows, CHUNK):
            n = min(CHUNK, zrows - off)
            pltpu.sync_copy(ones_v.at[pl.ds(0, n)], acc_sh.at[pl.ds(zb + off, n)])
        _fill_f32(ones_v, CHUNK, 16, 1.0)
        pltpu.sync_copy(dst_hbm.at[wid], dst_v)
        plsc.subcore_barrier()

        @pl.loop(0, K)
        def _(j):
            pltpu.sync_copy(ones_v, acc_sh.at[dst_v.at[j]], add=True)

        plsc.subcore_barrier()
        for off in range(0, zrows, CHUNK):
            n = min(CHUNK, zrows - off)
            pltpu.sync_copy(acc_sh.at[pl.ds(zb + off, n)],
                            out_hbm.at[c, pl.ds(zb + off, n)])

    return deg_kernel


def _make_agg_kernel(N, NACC, K, D):
    mesh = plsc.VectorSubcoreMesh(core_axis_name="c", subcore_axis_name="s")
    zrows = NACC // NS

    @functools.partial(
        pl.kernel,
        out_type=jax.ShapeDtypeStruct((NC, NACC, D), jnp.float32),
        mesh=mesh,
        scratch_types=[
            pltpu.VMEM((K, CHUNK), jnp.int32),
            pltpu.VMEM((K, CHUNK), jnp.int32),
            pltpu.VMEM((CHUNK, D), jnp.float32),
            pltpu.VMEM_SHARED((NACC, D), jnp.float32),
            pltpu.SemaphoreType.DMA,
        ],
    )
    def agg_kernel(table_hbm, src_hbm, dst_hbm, out_hbm,
                   src_v, dst_v, rows_v, acc_sh, sem):
        c = lax.axis_index("c")
        s = lax.axis_index("s")
        wid = s * NC + c
        _fill_f32(rows_v, CHUNK, D, 0.0)
        zb = s * zrows
        for off in range(0, zrows, CHUNK):
            n = min(CHUNK, zrows - off)
            pltpu.sync_copy(rows_v.at[pl.ds(0, n)], acc_sh.at[pl.ds(zb + off, n)])
        pltpu.sync_copy(src_hbm.at[wid], src_v)
        pltpu.sync_copy(dst_hbm.at[wid], dst_v)
        plsc.subcore_barrier()

        @pl.loop(0, K)
        def _(j):
            pltpu.async_copy(table_hbm.at[src_v.at[j]], rows_v, sem).wait()
            pltpu.sync_copy(rows_v, acc_sh.at[dst_v.at[j]], add=True)

        plsc.subcore_barrier()
        for off in range(0, zrows, CHUNK):
            n = min(CHUNK, zrows - off)
            pltpu.sync_copy(acc_sh.at[pl.ds(zb + off, n)],
                            out_hbm.at[c, pl.ds(zb + off, n)])

    return agg_kernel


def _mm_body(x_ref, w_ref, o_ref):
    o_ref[...] = jnp.dot(x_ref[...], w_ref[...],
                         preferred_element_type=jnp.float32)


def _dinv(d0_ref, d1_ref):
    deg = d0_ref[:, :1] + d1_ref[:, :1] + 1.0
    return lax.rsqrt(deg)


def _scale_body(z_ref, d0_ref, d1_ref, o_ref):
    o_ref[...] = z_ref[...] * _dinv(d0_ref, d1_ref)


def _layer2_body(a0_ref, a1_ref, p_ref, d0_ref, d1_ref, w_ref, b_ref, o_ref):
    dinv = _dinv(d0_ref, d1_ref)
    h = (a0_ref[...] + a1_ref[...] + p_ref[...]) * dinv + b_ref[...]
    h = jnp.maximum(h, 0.0)
    o_ref[...] = jnp.dot(h, w_ref[...],
                         preferred_element_type=jnp.float32) * dinv


def _final_body(a0_ref, a1_ref, p_ref, d0_ref, d1_ref, b_ref, o_ref):
    dinv = _dinv(d0_ref, d1_ref)
    o_ref[...] = (a0_ref[...] + a1_ref[...] + p_ref[...]) * dinv + b_ref[...]


def kernel(x, edge_index, W1, b1, W2, b2):
    N, D = x.shape
    E = edge_index.shape[1]
    src = edge_index[0].astype(jnp.int32)
    dst = edge_index[1].astype(jnp.int32)

    per_op = NW * CHUNK
    K = -(-E // per_op)
    padn = K * per_op - E
    NACC = -(-N // (8 * NS)) * (8 * NS) + 8 * NS
    src3 = jnp.concatenate([src, jnp.zeros((padn,), jnp.int32)]).reshape(NW, K, CHUNK)
    dst3 = jnp.concatenate([dst, jnp.full((padn,), N, jnp.int32)]).reshape(NW, K, CHUNK)

    deg_k = _make_deg_kernel(N, NACC, K)
    agg_k = _make_agg_kernel(N, NACC, K, D)

    degp = deg_k(dst3)
    d0, d1 = degp[0, :N], degp[1, :N]

    BR = 400
    G = N // BR
    fullW = pl.BlockSpec((D, D), lambda i: (0, 0))
    fullB = pl.BlockSpec((1, D), lambda i: (0, 0))
    rowD = lambda: pl.BlockSpec((BR, D), lambda i: (i, 0))
    rowd = lambda: pl.BlockSpec((BR, 16), lambda i: (i, 0))
    outD = jax.ShapeDtypeStruct((N, D), jnp.float32)
    b1r = b1.reshape(1, D)
    b2r = b2.reshape(1, D)

    z1 = pl.pallas_call(
        _mm_body, grid=(G,),
        in_specs=[rowD(), fullW],
        out_specs=rowD(), out_shape=outD)(x, W1)

    p1 = pl.pallas_call(
        _scale_body, grid=(G,),
        in_specs=[rowD(), rowd(), rowd()],
        out_specs=rowD(), out_shape=outD)(z1, d0, d1)

    a1 = agg_k(p1, src3, dst3)

    p2 = pl.pallas_call(
        _layer2_body, grid=(G,),
        in_specs=[rowD(), rowD(), rowD(), rowd(), rowd(), fullW, fullB],
        out_specs=rowD(), out_shape=outD)(a1[0, :N], a1[1, :N], p1, d0, d1, W2, b1r)

    a2 = agg_k(p2, src3, dst3)

    out = pl.pallas_call(
        _final_body, grid=(G,),
        in_specs=[rowD(), rowD(), rowD(), rowd(), rowd(), fullB],
        out_specs=rowD(), out_shape=outD)(a2[0, :N], a2[1, :N], p2, d0, d1, b2r)

    return out
